# baseline (speedup 1.0000x reference)
import functools

import jax
import jax.numpy as jnp
from jax import lax
from jax.experimental import pallas as pl
from jax.experimental.pallas import tpu as pltpu
from jax.experimental.pallas import tpu_sc as plsc

VOCAB = 100000
HIDDEN = 1024
B = 4
S = 2048
N = B * S
NC = 2
NS = 16
NW = NC * NS
ROWS_PER_W = N // NW
CHUNK = 32
NCHUNK = ROWS_PER_W // CHUNK
NSLICE = HIDDEN // 16
WPB = S // ROWS_PER_W
EPS = 1e-3

_GDN = lax.GatherDimensionNumbers(
    offset_dims=(), collapsed_slice_dims=(0,), start_index_map=(0,))


def _permute(x, idx):
    return lax.gather(x, idx.reshape(16, 1), _GDN, (1,),
                      mode=lax.GatherScatterMode.PROMISE_IN_BOUNDS)


def _hsum(x, perms):
    for p in perms:
        x = x + _permute(x, p)
    return x


def _rsqrt16(x):
    y = jnp.full((16,), 10.0 ** 6.0, jnp.float32)
    for k in range(-23, 25):
        thr = 10.0 ** (k / 2.0)
        guess = 10.0 ** (-(k + 0.5) / 4.0)
        y = jnp.where(x >= thr, guess, y)
    for _ in range(4):
        y = y * (1.5 - 0.5 * x * y * y)
    return y


@functools.partial(
    pl.kernel,
    mesh=plsc.VectorSubcoreMesh(core_axis_name="c", subcore_axis_name="s"),
    out_type=jax.ShapeDtypeStruct((N, HIDDEN), jnp.float32),
    scratch_types=[
        pltpu.VMEM((ROWS_PER_W,), jnp.int32),
        pltpu.VMEM((ROWS_PER_W,), jnp.int32),
        pltpu.VMEM((2 * HIDDEN,), jnp.float32),
        pltpu.VMEM((HIDDEN,), jnp.float32),
        pltpu.VMEM((HIDDEN,), jnp.float32),
        pltpu.VMEM((CHUNK, HIDDEN), jnp.float32),
        pltpu.VMEM((CHUNK, HIDDEN), jnp.float32),
        pltpu.SemaphoreType.DMA,
    ],
)
def _emb(tok_hbm, seg_hbm, we_hbm, tte_hbm, pos_hbm, gam_hbm, bet_hbm,
         out_hbm, idx_v, seg_v, tte_v, gam_v, bet_v, pos_v, rows_v, sem):
    wid = lax.axis_index("s") * NC + lax.axis_index("c")
    base = wid * ROWS_PER_W
    p0 = (wid % WPB) * ROWS_PER_W
    pltpu.sync_copy(tok_hbm.at[pl.ds(base, ROWS_PER_W)], idx_v)
    pltpu.sync_copy(seg_hbm.at[pl.ds(base, ROWS_PER_W)], seg_v)
    pltpu.sync_copy(tte_hbm, tte_v)
    pltpu.sync_copy(gam_hbm, gam_v)
    pltpu.sync_copy(bet_hbm, bet_v)
    lane_iota = lax.broadcasted_iota(jnp.int32, (16,), 0)
    perms = [lane_iota ^ k for k in (8, 4, 2, 1)]

    def chunk_body(c, carry):
        row0 = c * CHUNK
        pltpu.async_copy(we_hbm.at[idx_v.at[pl.ds(row0, CHUNK)]], rows_v,
                         sem).wait()
        pltpu.sync_copy(pos_hbm.at[pl.ds(p0 + row0, CHUNK)], pos_v)

        def row_body(r, rcarry):
            gr = row0 + r
            seg16 = seg_v[pl.ds((gr // 16) * 16, 16)]
            segf = _permute(seg16, jnp.full((16,), gr % 16, jnp.int32)
                            ).astype(jnp.float32)
            acc = jnp.zeros((16,), jnp.float32)
            acc2 = jnp.zeros((16,), jnp.float32)
            for i in range(NSLICE):
                sl = pl.ds(i * 16, 16)
                t0 = tte_v[sl]
                t = t0 + segf * (tte_v[pl.ds(HIDDEN + i * 16, 16)] - t0)
                v = rows_v[r, sl] + pos_v[r, sl] + t
                rows_v[r, sl] = v
                acc = acc + v
                acc2 = acc2 + v * v
            mv = _hsum(acc, perms) * (1.0 / HIDDEN)
            var = _hsum(acc2, perms) * (1.0 / HIDDEN) - mv * mv + EPS
            rstd = _rsqrt16(var)
            for i in range(NSLICE):
                sl = pl.ds(i * 16, 16)
                w = (rows_v[r, sl] - mv) * rstd
                rows_v[r, sl] = w * gam_v[sl] + bet_v[sl]
            return rcarry

        lax.fori_loop(0, CHUNK, row_body, 0)
        pltpu.sync_copy(rows_v, out_hbm.at[pl.ds(base + row0, CHUNK)])
        return carry

    lax.fori_loop(0, NCHUNK, chunk_body, 0)


def kernel(token, segment, word_embeddings, token_type_embeddings,
           position_embeddings, ln_gamma, ln_beta):
    tok = token.reshape(N).astype(jnp.int32)
    seg = segment.reshape(N).astype(jnp.int32)
    out = _emb(tok, seg, word_embeddings,
               token_type_embeddings.reshape(2 * HIDDEN),
               position_embeddings, ln_gamma, ln_beta)
    return out.reshape(B, S, HIDDEN)


# speedup vs baseline: 1.0612x; 1.0612x over previous
import functools

import jax
import jax.numpy as jnp
from jax import lax
from jax.experimental import pallas as pl
from jax.experimental.pallas import tpu as pltpu
from jax.experimental.pallas import tpu_sc as plsc

VOCAB = 100000
HIDDEN = 1024
B = 4
S = 2048
N = B * S
NC = 2
NS = 16
NW = NC * NS
ROWS_PER_W = N // NW
CHUNK = 16
NCHUNK = ROWS_PER_W // CHUNK
NSLICE = HIDDEN // 16
WPB = S // ROWS_PER_W
EPS = 1e-3

_GDN = lax.GatherDimensionNumbers(
    offset_dims=(), collapsed_slice_dims=(0,), start_index_map=(0,))


def _permute(x, idx):
    return lax.gather(x, idx.reshape(16, 1), _GDN, (1,),
                      mode=lax.GatherScatterMode.PROMISE_IN_BOUNDS)


def _hsum(x, perms):
    for p in perms:
        x = x + _permute(x, p)
    return x


def _rsqrt16(x):
    y = jnp.full((16,), 10.0 ** 6.0, jnp.float32)
    for k in range(-23, 25):
        thr = 10.0 ** (k / 2.0)
        guess = 10.0 ** (-(k + 0.5) / 4.0)
        y = jnp.where(x >= thr, guess, y)
    for _ in range(4):
        y = y * (1.5 - 0.5 * x * y * y)
    return y


@functools.partial(
    pl.kernel,
    mesh=plsc.VectorSubcoreMesh(core_axis_name="c", subcore_axis_name="s"),
    out_type=jax.ShapeDtypeStruct((N, HIDDEN), jnp.float32),
    scratch_types=[
        pltpu.VMEM((ROWS_PER_W,), jnp.int32),
        pltpu.VMEM((ROWS_PER_W,), jnp.int32),
        pltpu.VMEM((2 * HIDDEN,), jnp.float32),
        pltpu.VMEM((HIDDEN,), jnp.float32),
        pltpu.VMEM((HIDDEN,), jnp.float32),
        pltpu.VMEM((CHUNK, HIDDEN), jnp.float32),
        pltpu.VMEM((CHUNK, HIDDEN), jnp.float32),
        pltpu.VMEM((CHUNK, HIDDEN), jnp.float32),
        pltpu.VMEM((CHUNK, HIDDEN), jnp.float32),
        pltpu.SemaphoreType.DMA,
        pltpu.SemaphoreType.DMA,
        pltpu.SemaphoreType.DMA,
        pltpu.SemaphoreType.DMA,
        pltpu.SemaphoreType.DMA,
        pltpu.SemaphoreType.DMA,
    ],
)
def _emb(tok_hbm, seg_hbm, we_hbm, tte_hbm, pos_hbm, gam_hbm, bet_hbm,
         out_hbm, idx_v, seg_v, tte_v, gam_v, bet_v, rows0, rows1,
         pos0, pos1, sg0, sg1, sp0, sp1, so0, so1):
    wid = lax.axis_index("s") * NC + lax.axis_index("c")
    base = wid * ROWS_PER_W
    p0 = (wid % WPB) * ROWS_PER_W
    pltpu.sync_copy(tok_hbm.at[pl.ds(base, ROWS_PER_W)], idx_v)
    pltpu.sync_copy(seg_hbm.at[pl.ds(base, ROWS_PER_W)], seg_v)
    pltpu.sync_copy(tte_hbm, tte_v)
    pltpu.sync_copy(gam_hbm, gam_v)
    pltpu.sync_copy(bet_hbm, bet_v)
    lane_iota = lax.broadcasted_iota(jnp.int32, (16,), 0)
    perms = [lane_iota ^ k for k in (8, 4, 2, 1)]

    rows = (rows0, rows1)
    poss = (pos0, pos1)
    sg = (sg0, sg1)
    sp = (sp0, sp1)
    so = (so0, so1)

    def start_in(c, b):
        pltpu.async_copy(we_hbm.at[idx_v.at[pl.ds(c * CHUNK, CHUNK)]],
                         rows[b], sg[b])
        pltpu.async_copy(pos_hbm.at[pl.ds(p0 + c * CHUNK, CHUNK)],
                         poss[b], sp[b])

    def wait_in(c, b):
        pltpu.make_async_copy(we_hbm.at[idx_v.at[pl.ds(c * CHUNK, CHUNK)]],
                              rows[b], sg[b]).wait()
        pltpu.make_async_copy(pos_hbm.at[pl.ds(p0 + c * CHUNK, CHUNK)],
                              poss[b], sp[b]).wait()

    def out_slice(c):
        return out_hbm.at[pl.ds(base + c * CHUNK, CHUNK)]

    def compute(c, b):
        rows_v, pos_v = rows[b], poss[b]
        row0 = c * CHUNK

        def row_body(r, rcarry):
            gr = row0 + r
            seg16 = seg_v[pl.ds((gr // 16) * 16, 16)]
            segf = _permute(seg16, jnp.full((16,), gr % 16, jnp.int32)
                            ).astype(jnp.float32)
            acc = jnp.zeros((16,), jnp.float32)
            acc2 = jnp.zeros((16,), jnp.float32)
            for i in range(NSLICE):
                sl = pl.ds(i * 16, 16)
                t0 = tte_v[sl]
                t = t0 + segf * (tte_v[pl.ds(HIDDEN + i * 16, 16)] - t0)
                v = rows_v[r, sl] + pos_v[r, sl] + t
                rows_v[r, sl] = v
                acc = acc + v
                acc2 = acc2 + v * v
            mv = _hsum(acc, perms) * (1.0 / HIDDEN)
            var = _hsum(acc2, perms) * (1.0 / HIDDEN) - mv * mv + EPS
            rstd = _rsqrt16(var)
            for i in range(NSLICE):
                sl = pl.ds(i * 16, 16)
                w = (rows_v[r, sl] - mv) * rstd
                rows_v[r, sl] = w * gam_v[sl] + bet_v[sl]
            return rcarry

        lax.fori_loop(0, CHUNK, row_body, 0)

    start_in(0, 0)

    def pair_body(pair, carry):
        for b in (0, 1):
            c = pair * 2 + b
            nb = 1 - b

            @pl.when(c >= 1)
            def _():
                pltpu.make_async_copy(rows[nb], out_slice(c - 1),
                                      so[nb]).wait()

            @pl.when(c + 1 < NCHUNK)
            def _():
                start_in(c + 1, nb)

            wait_in(c, b)
            compute(c, b)
            pltpu.async_copy(rows[b], out_slice(c), so[b])
        return carry

    lax.fori_loop(0, NCHUNK // 2, pair_body, 0)
    pltpu.make_async_copy(rows[1], out_slice(NCHUNK - 1), so[1]).wait()


def kernel(token, segment, word_embeddings, token_type_embeddings,
           position_embeddings, ln_gamma, ln_beta):
    tok = token.reshape(N).astype(jnp.int32)
    seg = segment.reshape(N).astype(jnp.int32)
    out = _emb(tok, seg, word_embeddings,
               token_type_embeddings.reshape(2 * HIDDEN),
               position_embeddings, ln_gamma, ln_beta)
    return out.reshape(B, S, HIDDEN)


# speedup vs baseline: 1.1155x; 1.0511x over previous
import functools

import jax
import jax.numpy as jnp
from jax import lax
from jax.experimental import pallas as pl
from jax.experimental.pallas import tpu as pltpu
from jax.experimental.pallas import tpu_sc as plsc

VOCAB = 100000
HIDDEN = 1024
B = 4
S = 2048
N = B * S
NC = 2
NS = 16
NW = NC * NS
ROWS_PER_W = N // NW
CHUNK = 16
NCHUNK = ROWS_PER_W // CHUNK
NSLICE = HIDDEN // 16
WPB = S // ROWS_PER_W
EPS = 1e-3

_GDN = lax.GatherDimensionNumbers(
    offset_dims=(), collapsed_slice_dims=(0,), start_index_map=(0,))


def _permute(x, idx):
    return lax.gather(x, idx.reshape(16, 1), _GDN, (1,),
                      mode=lax.GatherScatterMode.PROMISE_IN_BOUNDS)


def _hsum(x, perms):
    for p in perms:
        x = x + _permute(x, p)
    return x


def _rsqrt16(x):
    y = jnp.full((16,), 10.0 ** 1.75, jnp.float32)
    for k in range(-7, 8):
        thr = 10.0 ** (k / 2.0)
        guess = 10.0 ** (-(k + 0.5) / 4.0)
        y = jnp.where(x >= thr, guess, y)
    for _ in range(4):
        y = y * (1.5 - 0.5 * x * y * y)
    return y


@functools.partial(
    pl.kernel,
    mesh=plsc.VectorSubcoreMesh(core_axis_name="c", subcore_axis_name="s"),
    out_type=jax.ShapeDtypeStruct((N, HIDDEN), jnp.float32),
    scratch_types=(
        [
            pltpu.VMEM((ROWS_PER_W,), jnp.int32),
            pltpu.VMEM((ROWS_PER_W,), jnp.int32),
            pltpu.VMEM((HIDDEN,), jnp.float32),
            pltpu.VMEM((HIDDEN,), jnp.float32),
            pltpu.VMEM((2, HIDDEN), jnp.float32),
        ]
        + [pltpu.VMEM((CHUNK, HIDDEN), jnp.float32)] * 6
        + [pltpu.SemaphoreType.DMA] * 6
    ),
)
def _emb(tok_hbm, seg_hbm, we_hbm, tte_hbm, pos_hbm, gam_hbm, bet_hbm,
         out_hbm, idx_v, seg_v, gam_v, bet_v, tte_v,
         rb0, rb1, pb0, pb1, ob0, ob1, sg0, sg1, sp0, sp1, so0, so1):
    rbufs = (rb0, rb1)
    pbufs = (pb0, pb1)
    obufs = (ob0, ob1)
    sg = (sg0, sg1)
    sp = (sp0, sp1)
    so = (so0, so1)

    wid = lax.axis_index("s") * NC + lax.axis_index("c")
    base = wid * ROWS_PER_W
    p0 = (wid % WPB) * ROWS_PER_W
    pltpu.sync_copy(tok_hbm.at[pl.ds(base, ROWS_PER_W)], idx_v)
    pltpu.sync_copy(seg_hbm.at[pl.ds(base, ROWS_PER_W)], seg_v)
    pltpu.sync_copy(gam_hbm, gam_v)
    pltpu.sync_copy(bet_hbm, bet_v)
    pltpu.sync_copy(tte_hbm, tte_v)
    lane_iota = lax.broadcasted_iota(jnp.int32, (16,), 0)
    perms = [lane_iota ^ k for k in (8, 4, 2, 1)]

    def in_descs(c, b):
        sl = pl.ds(c * CHUNK, CHUNK)
        return (pltpu.make_async_copy(we_hbm.at[idx_v.at[sl]],
                                      rbufs[b], sg[b]),
                pltpu.make_async_copy(pos_hbm.at[pl.ds(p0 + c * CHUNK,
                                                       CHUNK)],
                                      pbufs[b], sp[b]))

    def start_in(c, b):
        sl = pl.ds(c * CHUNK, CHUNK)
        pltpu.async_copy(we_hbm.at[idx_v.at[sl]], rbufs[b], sg[b])
        pltpu.async_copy(pos_hbm.at[pl.ds(p0 + c * CHUNK, CHUNK)],
                         pbufs[b], sp[b])

    def wait_in(c, b):
        g, p = in_descs(c, b)
        g.wait()
        p.wait()

    def out_desc(c, ob):
        return pltpu.make_async_copy(
            obufs[ob], out_hbm.at[pl.ds(base + c * CHUNK, CHUNK)], so[ob])

    def compute(c, b):
        rows_v, pos_v, out_v = rbufs[b], pbufs[b], obufs[b]

        def row_body(r, rcarry):
            gr = c * CHUNK + r
            seg16 = seg_v[pl.ds((gr // 16) * 16, 16)]
            segf = _permute(seg16, jnp.full((16,), gr % 16, jnp.int32)
                            ).astype(jnp.float32)
            accs = [jnp.zeros((16,), jnp.float32) for _ in range(4)]
            acc2s = [jnp.zeros((16,), jnp.float32) for _ in range(4)]
            for i in range(NSLICE):
                sl = pl.ds(i * 16, 16)
                t0 = tte_v[0, sl]
                t = t0 + segf * (tte_v[1, sl] - t0)
                v = rows_v[r, sl] + pos_v[r, sl] + t
                out_v[r, sl] = v
                accs[i % 4] = accs[i % 4] + v
                acc2s[i % 4] = acc2s[i % 4] + v * v
            acc = (accs[0] + accs[1]) + (accs[2] + accs[3])
            acc2 = (acc2s[0] + acc2s[1]) + (acc2s[2] + acc2s[3])
            mv = _hsum(acc, perms) * (1.0 / HIDDEN)
            var = _hsum(acc2, perms) * (1.0 / HIDDEN) - mv * mv + EPS
            rstd = _rsqrt16(var)
            nms = -(mv * rstd)
            for i in range(NSLICE):
                sl = pl.ds(i * 16, 16)
                w = out_v[r, sl] * rstd + nms
                out_v[r, sl] = w * gam_v[sl] + bet_v[sl]
            return rcarry

        lax.fori_loop(0, CHUNK, row_body, 0)

    start_in(0, 0)

    def pair_body(pair, carry):
        for b in (0, 1):
            c = pair * 2 + b

            @pl.when(c + 1 < NCHUNK)
            def _():
                start_in(c + 1, 1 - b)

            wait_in(c, b)

            @pl.when(c >= 2)
            def _():
                out_desc(c - 2, b).wait()

            compute(c, b)
            pltpu.async_copy(obufs[b],
                             out_hbm.at[pl.ds(base + c * CHUNK, CHUNK)],
                             so[b])
        return carry

    lax.fori_loop(0, NCHUNK // 2, pair_body, 0)
    out_desc(NCHUNK - 2, 0).wait()
    out_desc(NCHUNK - 1, 1).wait()


def kernel(token, segment, word_embeddings, token_type_embeddings,
           position_embeddings, ln_gamma, ln_beta):
    tok = token.reshape(N).astype(jnp.int32)
    seg = segment.reshape(N).astype(jnp.int32)
    out = _emb(tok, seg, word_embeddings, token_type_embeddings,
               position_embeddings, ln_gamma, ln_beta)
    return out.reshape(B, S, HIDDEN)


# speedup vs baseline: 3.6286x; 3.2530x over previous
import functools

import jax
import jax.numpy as jnp
from jax import lax
from jax.experimental import pallas as pl
from jax.experimental.pallas import tpu as pltpu
from jax.experimental.pallas import tpu_sc as plsc

VOCAB = 100000
HIDDEN = 1024
B = 4
S = 2048
N = B * S
NC = 2
NS = 16
NW = NC * NS
ROWS_PER_W = N // NW
CHUNK = 32
NCHUNK = ROWS_PER_W // CHUNK
EPS = 1e-3
TBLK = 256


@functools.partial(
    pl.kernel,
    mesh=plsc.VectorSubcoreMesh(core_axis_name="c", subcore_axis_name="s"),
    out_type=jax.ShapeDtypeStruct((N, HIDDEN), jnp.float32),
    scratch_types=[
        pltpu.VMEM((ROWS_PER_W,), jnp.int32),
        pltpu.VMEM((CHUNK, HIDDEN), jnp.float32),
        pltpu.VMEM((CHUNK, HIDDEN), jnp.float32),
        pltpu.SemaphoreType.DMA,
        pltpu.SemaphoreType.DMA,
        pltpu.SemaphoreType.DMA,
        pltpu.SemaphoreType.DMA,
    ],
)
def _gather_sc(tok_hbm, we_hbm, out_hbm, idx_v, rb0, rb1,
               sg0, sg1, so0, so1):
    rbufs = (rb0, rb1)
    sg = (sg0, sg1)
    so = (so0, so1)

    wid = lax.axis_index("s") * NC + lax.axis_index("c")
    base = wid * ROWS_PER_W
    pltpu.sync_copy(tok_hbm.at[pl.ds(base, ROWS_PER_W)], idx_v)

    def g_desc(c, b):
        return pltpu.make_async_copy(
            we_hbm.at[idx_v.at[pl.ds(c * CHUNK, CHUNK)]], rbufs[b], sg[b])

    def w_desc(c, b):
        return pltpu.make_async_copy(
            rbufs[b], out_hbm.at[pl.ds(base + c * CHUNK, CHUNK)], so[b])

    def start_gather(c, b):
        pltpu.async_copy(we_hbm.at[idx_v.at[pl.ds(c * CHUNK, CHUNK)]],
                         rbufs[b], sg[b])

    def start_wb(c, b):
        pltpu.async_copy(rbufs[b],
                         out_hbm.at[pl.ds(base + c * CHUNK, CHUNK)], so[b])

    def pair_body(pair, carry):
        for b in (0, 1):
            c = pair * 2 + b

            @pl.when(c >= 2)
            def _():
                w_desc(c - 2, b).wait()

            start_gather(c, b)

            @pl.when(c >= 1)
            def _():
                g_desc(c - 1, 1 - b).wait()
                start_wb(c - 1, 1 - b)
        return carry

    lax.fori_loop(0, NCHUNK // 2, pair_body, 0)
    g_desc(NCHUNK - 1, 1).wait()
    start_wb(NCHUNK - 1, 1)
    w_desc(NCHUNK - 2, 0).wait()
    w_desc(NCHUNK - 1, 1).wait()


def _ln_body(g_ref, seg_ref, pos_ref, tte_ref, gam_ref, bet_ref, out_ref):
    g = g_ref[...]
    segf = seg_ref[...]
    t0 = tte_ref[0:1, :]
    dt = tte_ref[1:2, :] - t0
    v = g + pos_ref[...] + t0 + segf * dt
    mean = jnp.mean(v, axis=-1, keepdims=True)
    var = jnp.mean(v * v, axis=-1, keepdims=True) - mean * mean
    w = (v - mean) * lax.rsqrt(var + EPS)
    out_ref[...] = w * gam_ref[...] + bet_ref[...]


def _ln_tc(gathered, segf, pos, tte, gam2d, bet2d):
    nblk = N // TBLK
    wpb = S // TBLK
    return pl.pallas_call(
        _ln_body,
        grid=(nblk,),
        in_specs=[
            pl.BlockSpec((TBLK, HIDDEN), lambda k: (k, 0)),
            pl.BlockSpec((TBLK, 1), lambda k: (k, 0)),
            pl.BlockSpec((TBLK, HIDDEN), lambda k: (k % wpb, 0)),
            pl.BlockSpec((2, HIDDEN), lambda k: (0, 0)),
            pl.BlockSpec((1, HIDDEN), lambda k: (0, 0)),
            pl.BlockSpec((1, HIDDEN), lambda k: (0, 0)),
        ],
        out_specs=pl.BlockSpec((TBLK, HIDDEN), lambda k: (k, 0)),
        out_shape=jax.ShapeDtypeStruct((N, HIDDEN), jnp.float32),
    )(gathered, segf, pos, tte, gam2d, bet2d)


def kernel(token, segment, word_embeddings, token_type_embeddings,
           position_embeddings, ln_gamma, ln_beta):
    tok = token.reshape(N).astype(jnp.int32)
    segf = segment.reshape(N, 1).astype(jnp.float32)
    gathered = _gather_sc(tok, word_embeddings)
    out = _ln_tc(gathered, segf, position_embeddings,
                 token_type_embeddings, ln_gamma.reshape(1, HIDDEN),
                 ln_beta.reshape(1, HIDDEN))
    return out.reshape(B, S, HIDDEN)


# speedup vs baseline: 3.6398x; 1.0031x over previous
import functools

import jax
import jax.numpy as jnp
from jax import lax
from jax.experimental import pallas as pl
from jax.experimental.pallas import tpu as pltpu
from jax.experimental.pallas import tpu_sc as plsc

VOCAB = 100000
HIDDEN = 1024
B = 4
S = 2048
N = B * S
NC = 2
NS = 16
NW = NC * NS
ROWS_PER_W = N // NW
CHUNK = 32
NCHUNK = ROWS_PER_W // CHUNK
EPS = 1e-3
TBLK = 256


@functools.partial(
    pl.kernel,
    mesh=plsc.VectorSubcoreMesh(core_axis_name="c", subcore_axis_name="s"),
    out_type=jax.ShapeDtypeStruct((N, HIDDEN), jnp.float32),
    scratch_types=[
        pltpu.VMEM((ROWS_PER_W,), jnp.int32),
        pltpu.VMEM((CHUNK, HIDDEN), jnp.float32),
        pltpu.VMEM((CHUNK, HIDDEN), jnp.float32),
        pltpu.VMEM((CHUNK, HIDDEN), jnp.float32),
        pltpu.SemaphoreType.DMA,
        pltpu.SemaphoreType.DMA,
        pltpu.SemaphoreType.DMA,
        pltpu.SemaphoreType.DMA,
        pltpu.SemaphoreType.DMA,
        pltpu.SemaphoreType.DMA,
    ],
)
def _gather_sc(tok_hbm, we_hbm, out_hbm, idx_v, rb0, rb1, rb2,
               sg0, sg1, sg2, so0, so1, so2):
    rbufs = (rb0, rb1, rb2)
    sg = (sg0, sg1, sg2)
    so = (so0, so1, so2)

    wid = lax.axis_index("s") * NC + lax.axis_index("c")
    base = wid * ROWS_PER_W
    pltpu.sync_copy(tok_hbm.at[pl.ds(base, ROWS_PER_W)], idx_v)

    def g_desc(c, b):
        return pltpu.make_async_copy(
            we_hbm.at[idx_v.at[pl.ds(c * CHUNK, CHUNK)]], rbufs[b], sg[b])

    def w_desc(c, b):
        return pltpu.make_async_copy(
            rbufs[b], out_hbm.at[pl.ds(base + c * CHUNK, CHUNK)], so[b])

    def start_gather(c, b):
        pltpu.async_copy(we_hbm.at[idx_v.at[pl.ds(c * CHUNK, CHUNK)]],
                         rbufs[b], sg[b])

    def start_wb(c, b):
        pltpu.async_copy(rbufs[b],
                         out_hbm.at[pl.ds(base + c * CHUNK, CHUNK)], so[b])

    for c in range(NCHUNK):
        b = c % 3
        if c >= 3:
            w_desc(c - 3, b).wait()
        start_gather(c, b)
        if c >= 1:
            g_desc(c - 1, (c - 1) % 3).wait()
            start_wb(c - 1, (c - 1) % 3)
    last = NCHUNK - 1
    g_desc(last, last % 3).wait()
    start_wb(last, last % 3)
    for c in range(NCHUNK - 3, NCHUNK):
        w_desc(c, c % 3).wait()


def _ln_body(g_ref, seg_ref, pos_ref, tte_ref, gam_ref, bet_ref, out_ref):
    g = g_ref[...]
    segf = seg_ref[...]
    t0 = tte_ref[0:1, :]
    dt = tte_ref[1:2, :] - t0
    v = g + pos_ref[...] + t0 + segf * dt
    mean = jnp.mean(v, axis=-1, keepdims=True)
    var = jnp.mean(v * v, axis=-1, keepdims=True) - mean * mean
    w = (v - mean) * lax.rsqrt(var + EPS)
    out_ref[...] = w * gam_ref[...] + bet_ref[...]


def _ln_tc(gathered, segf, pos, tte, gam2d, bet2d):
    wpb = S // TBLK
    return pl.pallas_call(
        _ln_body,
        grid=(wpb, B),
        in_specs=[
            pl.BlockSpec((TBLK, HIDDEN), lambda j, b: (b * wpb + j, 0)),
            pl.BlockSpec((TBLK, 1), lambda j, b: (b * wpb + j, 0)),
            pl.BlockSpec((TBLK, HIDDEN), lambda j, b: (j, 0)),
            pl.BlockSpec((2, HIDDEN), lambda j, b: (0, 0)),
            pl.BlockSpec((1, HIDDEN), lambda j, b: (0, 0)),
            pl.BlockSpec((1, HIDDEN), lambda j, b: (0, 0)),
        ],
        out_specs=pl.BlockSpec((TBLK, HIDDEN), lambda j, b: (b * wpb + j, 0)),
        out_shape=jax.ShapeDtypeStruct((N, HIDDEN), jnp.float32),
    )(gathered, segf, pos, tte, gam2d, bet2d)


def kernel(token, segment, word_embeddings, token_type_embeddings,
           position_embeddings, ln_gamma, ln_beta):
    tok = token.reshape(N).astype(jnp.int32)
    segf = segment.reshape(N, 1).astype(jnp.float32)
    gathered = _gather_sc(tok, word_embeddings)
    out = _ln_tc(gathered, segf, position_embeddings,
                 token_type_embeddings, ln_gamma.reshape(1, HIDDEN),
                 ln_beta.reshape(1, HIDDEN))
    return out.reshape(B, S, HIDDEN)


# speedup vs baseline: 3.7473x; 1.0296x over previous
import functools

import jax
import jax.numpy as jnp
from jax import lax
from jax.experimental import pallas as pl
from jax.experimental.pallas import tpu as pltpu
from jax.experimental.pallas import tpu_sc as plsc

VOCAB = 100000
HIDDEN = 1024
B = 4
S = 2048
N = B * S
NC = 2
NS = 16
NW = NC * NS
ROWS_PER_W = N // NW
CHUNK = 32
NCHUNK = ROWS_PER_W // CHUNK
EPS = 1e-3
TBLK = 256


@functools.partial(
    pl.kernel,
    mesh=plsc.VectorSubcoreMesh(core_axis_name="c", subcore_axis_name="s"),
    out_type=jax.ShapeDtypeStruct((N, HIDDEN), jnp.float32),
    scratch_types=[
        pltpu.VMEM((ROWS_PER_W,), jnp.int32),
        pltpu.VMEM((CHUNK, HIDDEN), jnp.float32),
        pltpu.VMEM((CHUNK, HIDDEN), jnp.float32),
        pltpu.SemaphoreType.DMA,
        pltpu.SemaphoreType.DMA,
        pltpu.SemaphoreType.DMA,
        pltpu.SemaphoreType.DMA,
    ],
)
def _gather_sc(tok_hbm, we_hbm, out_hbm, idx_v, rb0, rb1,
               sg0, sg1, so0, so1):
    rbufs = (rb0, rb1)
    sg = (sg0, sg1)
    so = (so0, so1)

    wid = lax.axis_index("s") * NC + lax.axis_index("c")
    base = wid * ROWS_PER_W
    pltpu.sync_copy(tok_hbm.at[pl.ds(base, ROWS_PER_W)], idx_v)

    def g_desc(c, b):
        return pltpu.make_async_copy(
            we_hbm.at[idx_v.at[pl.ds(c * CHUNK, CHUNK)]], rbufs[b], sg[b])

    def w_desc(c, b):
        return pltpu.make_async_copy(
            rbufs[b], out_hbm.at[pl.ds(base + c * CHUNK, CHUNK)], so[b])

    def start_gather(c, b):
        pltpu.async_copy(we_hbm.at[idx_v.at[pl.ds(c * CHUNK, CHUNK)]],
                         rbufs[b], sg[b])

    def start_wb(c, b):
        pltpu.async_copy(rbufs[b],
                         out_hbm.at[pl.ds(base + c * CHUNK, CHUNK)], so[b])

    def pair_body(pair, carry):
        for b in (0, 1):
            c = pair * 2 + b

            @pl.when(c >= 2)
            def _():
                w_desc(c - 2, b).wait()

            start_gather(c, b)

            @pl.when(c >= 1)
            def _():
                g_desc(c - 1, 1 - b).wait()
                start_wb(c - 1, 1 - b)
        return carry

    lax.fori_loop(0, NCHUNK // 2, pair_body, 0)
    g_desc(NCHUNK - 1, 1).wait()
    start_wb(NCHUNK - 1, 1)
    w_desc(NCHUNK - 2, 0).wait()
    w_desc(NCHUNK - 1, 1).wait()


def _ln_body(g_ref, seg_ref, pos_ref, tte_ref, gam_ref, bet_ref, out_ref):
    g = g_ref[...]
    segf = seg_ref[...]
    t0 = tte_ref[0:1, :]
    dt = tte_ref[1:2, :] - t0
    v = g + pos_ref[...] + t0 + segf * dt
    mean = jnp.mean(v, axis=-1, keepdims=True)
    var = jnp.mean(v * v, axis=-1, keepdims=True) - mean * mean
    w = (v - mean) * lax.rsqrt(var + EPS)
    out_ref[...] = w * gam_ref[...] + bet_ref[...]


def _ln_tc(gathered, segf, pos, tte, gam2d, bet2d):
    wpb = S // TBLK
    return pl.pallas_call(
        _ln_body,
        grid=(wpb, B),
        in_specs=[
            pl.BlockSpec((TBLK, HIDDEN), lambda j, b: (b * wpb + j, 0)),
            pl.BlockSpec((TBLK, 1), lambda j, b: (b * wpb + j, 0)),
            pl.BlockSpec((TBLK, HIDDEN), lambda j, b: (j, 0)),
            pl.BlockSpec((2, HIDDEN), lambda j, b: (0, 0)),
            pl.BlockSpec((1, HIDDEN), lambda j, b: (0, 0)),
            pl.BlockSpec((1, HIDDEN), lambda j, b: (0, 0)),
        ],
        out_specs=pl.BlockSpec((TBLK, HIDDEN), lambda j, b: (b * wpb + j, 0)),
        out_shape=jax.ShapeDtypeStruct((N, HIDDEN), jnp.float32),
    )(gathered, segf, pos, tte, gam2d, bet2d)


def kernel(token, segment, word_embeddings, token_type_embeddings,
           position_embeddings, ln_gamma, ln_beta):
    tok = token.reshape(N).astype(jnp.int32)
    segf = segment.reshape(N, 1).astype(jnp.float32)
    gathered = _gather_sc(tok, word_embeddings)
    out = _ln_tc(gathered, segf, position_embeddings,
                 token_type_embeddings, ln_gamma.reshape(1, HIDDEN),
                 ln_beta.reshape(1, HIDDEN))
    return out.reshape(B, S, HIDDEN)


# speedup vs baseline: 4.1201x; 1.0995x over previous
import functools

import jax
import jax.numpy as jnp
from jax import lax
from jax.experimental import pallas as pl
from jax.experimental.pallas import tpu as pltpu
from jax.experimental.pallas import tpu_sc as plsc

VOCAB = 100000
HIDDEN = 1024
B = 4
S = 2048
N = B * S
NC = 2
NS = 16
NW = NC * NS
ROWS_PER_W = N // NW
CHUNK = 32
NCHUNK = ROWS_PER_W // CHUNK
EPS = 1e-3
TBLK = 512


@functools.partial(
    pl.kernel,
    mesh=plsc.VectorSubcoreMesh(core_axis_name="c", subcore_axis_name="s"),
    out_type=jax.ShapeDtypeStruct((N, HIDDEN), jnp.float32),
    scratch_types=[
        pltpu.VMEM((ROWS_PER_W,), jnp.int32),
        pltpu.VMEM((CHUNK, HIDDEN), jnp.float32),
        pltpu.VMEM((CHUNK, HIDDEN), jnp.float32),
        pltpu.SemaphoreType.DMA,
        pltpu.SemaphoreType.DMA,
        pltpu.SemaphoreType.DMA,
        pltpu.SemaphoreType.DMA,
    ],
)
def _gather_sc(tok_hbm, we_hbm, out_hbm, idx_v, rb0, rb1,
               sg0, sg1, so0, so1):
    rbufs = (rb0, rb1)
    sg = (sg0, sg1)
    so = (so0, so1)

    wid = lax.axis_index("s") * NC + lax.axis_index("c")
    base = wid * ROWS_PER_W
    pltpu.sync_copy(tok_hbm.at[pl.ds(base, ROWS_PER_W)], idx_v)

    def g_desc(c, b):
        return pltpu.make_async_copy(
            we_hbm.at[idx_v.at[pl.ds(c * CHUNK, CHUNK)]], rbufs[b], sg[b])

    def w_desc(c, b):
        return pltpu.make_async_copy(
            rbufs[b], out_hbm.at[pl.ds(base + c * CHUNK, CHUNK)], so[b])

    def start_gather(c, b):
        pltpu.async_copy(we_hbm.at[idx_v.at[pl.ds(c * CHUNK, CHUNK)]],
                         rbufs[b], sg[b])

    def start_wb(c, b):
        pltpu.async_copy(rbufs[b],
                         out_hbm.at[pl.ds(base + c * CHUNK, CHUNK)], so[b])

    def pair_body(pair, carry):
        for b in (0, 1):
            c = pair * 2 + b

            @pl.when(c >= 2)
            def _():
                w_desc(c - 2, b).wait()

            start_gather(c, b)

            @pl.when(c >= 1)
            def _():
                g_desc(c - 1, 1 - b).wait()
                start_wb(c - 1, 1 - b)
        return carry

    lax.fori_loop(0, NCHUNK // 2, pair_body, 0)
    g_desc(NCHUNK - 1, 1).wait()
    start_wb(NCHUNK - 1, 1)
    w_desc(NCHUNK - 2, 0).wait()
    w_desc(NCHUNK - 1, 1).wait()


def _ln_body(g_ref, seg_ref, pos_ref, tte_ref, gam_ref, bet_ref, out_ref):
    g = g_ref[...]
    segf = seg_ref[...]
    t0 = tte_ref[0:1, :]
    dt = tte_ref[1:2, :] - t0
    v = g + pos_ref[...] + t0 + segf * dt
    mean = jnp.mean(v, axis=-1, keepdims=True)
    var = jnp.mean(v * v, axis=-1, keepdims=True) - mean * mean
    w = (v - mean) * lax.rsqrt(var + EPS)
    out_ref[...] = w * gam_ref[...] + bet_ref[...]


def _ln_tc(gathered, segf, pos, tte, gam2d, bet2d):
    wpb = S // TBLK
    return pl.pallas_call(
        _ln_body,
        grid=(wpb, B),
        in_specs=[
            pl.BlockSpec((TBLK, HIDDEN), lambda j, b: (b * wpb + j, 0)),
            pl.BlockSpec((TBLK, 1), lambda j, b: (b * wpb + j, 0)),
            pl.BlockSpec((TBLK, HIDDEN), lambda j, b: (j, 0)),
            pl.BlockSpec((2, HIDDEN), lambda j, b: (0, 0)),
            pl.BlockSpec((1, HIDDEN), lambda j, b: (0, 0)),
            pl.BlockSpec((1, HIDDEN), lambda j, b: (0, 0)),
        ],
        out_specs=pl.BlockSpec((TBLK, HIDDEN), lambda j, b: (b * wpb + j, 0)),
        out_shape=jax.ShapeDtypeStruct((N, HIDDEN), jnp.float32),
    )(gathered, segf, pos, tte, gam2d, bet2d)


def kernel(token, segment, word_embeddings, token_type_embeddings,
           position_embeddings, ln_gamma, ln_beta):
    tok = token.reshape(N).astype(jnp.int32)
    segf = segment.reshape(N, 1).astype(jnp.float32)
    gathered = _gather_sc(tok, word_embeddings)
    out = _ln_tc(gathered, segf, position_embeddings,
                 token_type_embeddings, ln_gamma.reshape(1, HIDDEN),
                 ln_beta.reshape(1, HIDDEN))
    return out.reshape(B, S, HIDDEN)


# speedup vs baseline: 4.3549x; 1.0570x over previous
import functools

import jax
import jax.numpy as jnp
from jax import lax
from jax.experimental import pallas as pl
from jax.experimental.pallas import tpu as pltpu
from jax.experimental.pallas import tpu_sc as plsc

VOCAB = 100000
HIDDEN = 1024
B = 4
S = 2048
N = B * S
NC = 2
NS = 16
NW = NC * NS
ROWS_PER_W = N // NW
CHUNK = 32
NCHUNK = ROWS_PER_W // CHUNK
EPS = 1e-3
TBLK = 1024


@functools.partial(
    pl.kernel,
    mesh=plsc.VectorSubcoreMesh(core_axis_name="c", subcore_axis_name="s"),
    out_type=jax.ShapeDtypeStruct((N, HIDDEN), jnp.float32),
    scratch_types=[
        pltpu.VMEM((ROWS_PER_W,), jnp.int32),
        pltpu.VMEM((CHUNK, HIDDEN), jnp.float32),
        pltpu.VMEM((CHUNK, HIDDEN), jnp.float32),
        pltpu.SemaphoreType.DMA,
        pltpu.SemaphoreType.DMA,
        pltpu.SemaphoreType.DMA,
        pltpu.SemaphoreType.DMA,
    ],
)
def _gather_sc(tok_hbm, we_hbm, out_hbm, idx_v, rb0, rb1,
               sg0, sg1, so0, so1):
    rbufs = (rb0, rb1)
    sg = (sg0, sg1)
    so = (so0, so1)

    wid = lax.axis_index("s") * NC + lax.axis_index("c")
    base = wid * ROWS_PER_W
    pltpu.sync_copy(tok_hbm.at[pl.ds(base, ROWS_PER_W)], idx_v)

    def g_desc(c, b):
        return pltpu.make_async_copy(
            we_hbm.at[idx_v.at[pl.ds(c * CHUNK, CHUNK)]], rbufs[b], sg[b])

    def w_desc(c, b):
        return pltpu.make_async_copy(
            rbufs[b], out_hbm.at[pl.ds(base + c * CHUNK, CHUNK)], so[b])

    def start_gather(c, b):
        pltpu.async_copy(we_hbm.at[idx_v.at[pl.ds(c * CHUNK, CHUNK)]],
                         rbufs[b], sg[b])

    def start_wb(c, b):
        pltpu.async_copy(rbufs[b],
                         out_hbm.at[pl.ds(base + c * CHUNK, CHUNK)], so[b])

    def pair_body(pair, carry):
        for b in (0, 1):
            c = pair * 2 + b

            @pl.when(c >= 2)
            def _():
                w_desc(c - 2, b).wait()

            start_gather(c, b)

            @pl.when(c >= 1)
            def _():
                g_desc(c - 1, 1 - b).wait()
                start_wb(c - 1, 1 - b)
        return carry

    lax.fori_loop(0, NCHUNK // 2, pair_body, 0)
    g_desc(NCHUNK - 1, 1).wait()
    start_wb(NCHUNK - 1, 1)
    w_desc(NCHUNK - 2, 0).wait()
    w_desc(NCHUNK - 1, 1).wait()


def _ln_body(g_ref, seg_ref, pos_ref, tte_ref, gam_ref, bet_ref, out_ref):
    g = g_ref[...]
    segf = seg_ref[...]
    t0 = tte_ref[0:1, :]
    dt = tte_ref[1:2, :] - t0
    v = g + pos_ref[...] + t0 + segf * dt
    mean = jnp.mean(v, axis=-1, keepdims=True)
    var = jnp.mean(v * v, axis=-1, keepdims=True) - mean * mean
    w = (v - mean) * lax.rsqrt(var + EPS)
    out_ref[...] = w * gam_ref[...] + bet_ref[...]


def _ln_tc(gathered, segf, pos, tte, gam2d, bet2d):
    wpb = S // TBLK
    return pl.pallas_call(
        _ln_body,
        grid=(wpb, B),
        in_specs=[
            pl.BlockSpec((TBLK, HIDDEN), lambda j, b: (b * wpb + j, 0)),
            pl.BlockSpec((TBLK, 1), lambda j, b: (b * wpb + j, 0)),
            pl.BlockSpec((TBLK, HIDDEN), lambda j, b: (j, 0)),
            pl.BlockSpec((2, HIDDEN), lambda j, b: (0, 0)),
            pl.BlockSpec((1, HIDDEN), lambda j, b: (0, 0)),
            pl.BlockSpec((1, HIDDEN), lambda j, b: (0, 0)),
        ],
        out_specs=pl.BlockSpec((TBLK, HIDDEN), lambda j, b: (b * wpb + j, 0)),
        out_shape=jax.ShapeDtypeStruct((N, HIDDEN), jnp.float32),
    )(gathered, segf, pos, tte, gam2d, bet2d)


def kernel(token, segment, word_embeddings, token_type_embeddings,
           position_embeddings, ln_gamma, ln_beta):
    tok = token.reshape(N).astype(jnp.int32)
    segf = segment.reshape(N, 1).astype(jnp.float32)
    gathered = _gather_sc(tok, word_embeddings)
    out = _ln_tc(gathered, segf, position_embeddings,
                 token_type_embeddings, ln_gamma.reshape(1, HIDDEN),
                 ln_beta.reshape(1, HIDDEN))
    return out.reshape(B, S, HIDDEN)


# speedup vs baseline: 4.3988x; 1.0101x over previous
import functools

import jax
import jax.numpy as jnp
from jax import lax
from jax.experimental import pallas as pl
from jax.experimental.pallas import tpu as pltpu
from jax.experimental.pallas import tpu_sc as plsc

VOCAB = 100000
HIDDEN = 1024
B = 4
S = 2048
N = B * S
NC = 2
NS = 16
NW = NC * NS
ROWS_PER_W = N // NW
CHUNK = 32
NCHUNK = ROWS_PER_W // CHUNK
EPS = 1e-3
TBLK = 2048


@functools.partial(
    pl.kernel,
    mesh=plsc.VectorSubcoreMesh(core_axis_name="c", subcore_axis_name="s"),
    out_type=jax.ShapeDtypeStruct((N, HIDDEN), jnp.float32),
    scratch_types=[
        pltpu.VMEM((ROWS_PER_W,), jnp.int32),
        pltpu.VMEM((CHUNK, HIDDEN), jnp.float32),
        pltpu.VMEM((CHUNK, HIDDEN), jnp.float32),
        pltpu.SemaphoreType.DMA,
        pltpu.SemaphoreType.DMA,
        pltpu.SemaphoreType.DMA,
        pltpu.SemaphoreType.DMA,
    ],
)
def _gather_sc(tok_hbm, we_hbm, out_hbm, idx_v, rb0, rb1,
               sg0, sg1, so0, so1):
    rbufs = (rb0, rb1)
    sg = (sg0, sg1)
    so = (so0, so1)

    wid = lax.axis_index("s") * NC + lax.axis_index("c")
    base = wid * ROWS_PER_W
    pltpu.sync_copy(tok_hbm.at[pl.ds(base, ROWS_PER_W)], idx_v)

    def g_desc(c, b):
        return pltpu.make_async_copy(
            we_hbm.at[idx_v.at[pl.ds(c * CHUNK, CHUNK)]], rbufs[b], sg[b])

    def w_desc(c, b):
        return pltpu.make_async_copy(
            rbufs[b], out_hbm.at[pl.ds(base + c * CHUNK, CHUNK)], so[b])

    def start_gather(c, b):
        pltpu.async_copy(we_hbm.at[idx_v.at[pl.ds(c * CHUNK, CHUNK)]],
                         rbufs[b], sg[b])

    def start_wb(c, b):
        pltpu.async_copy(rbufs[b],
                         out_hbm.at[pl.ds(base + c * CHUNK, CHUNK)], so[b])

    def pair_body(pair, carry):
        for b in (0, 1):
            c = pair * 2 + b

            @pl.when(c >= 2)
            def _():
                w_desc(c - 2, b).wait()

            start_gather(c, b)

            @pl.when(c >= 1)
            def _():
                g_desc(c - 1, 1 - b).wait()
                start_wb(c - 1, 1 - b)
        return carry

    lax.fori_loop(0, NCHUNK // 2, pair_body, 0)
    g_desc(NCHUNK - 1, 1).wait()
    start_wb(NCHUNK - 1, 1)
    w_desc(NCHUNK - 2, 0).wait()
    w_desc(NCHUNK - 1, 1).wait()


def _ln_body(g_ref, seg_ref, pos_ref, tte_ref, gam_ref, bet_ref, out_ref):
    g = g_ref[...]
    segf = seg_ref[...]
    t0 = tte_ref[0:1, :]
    dt = tte_ref[1:2, :] - t0
    v = g + pos_ref[...] + t0 + segf * dt
    mean = jnp.mean(v, axis=-1, keepdims=True)
    var = jnp.mean(v * v, axis=-1, keepdims=True) - mean * mean
    w = (v - mean) * lax.rsqrt(var + EPS)
    out_ref[...] = w * gam_ref[...] + bet_ref[...]


def _ln_tc(gathered, segf, pos, tte, gam2d, bet2d):
    wpb = S // TBLK
    return pl.pallas_call(
        _ln_body,
        grid=(wpb, B),
        in_specs=[
            pl.BlockSpec((TBLK, HIDDEN), lambda j, b: (b * wpb + j, 0)),
            pl.BlockSpec((TBLK, 1), lambda j, b: (b * wpb + j, 0)),
            pl.BlockSpec((TBLK, HIDDEN), lambda j, b: (j, 0)),
            pl.BlockSpec((2, HIDDEN), lambda j, b: (0, 0)),
            pl.BlockSpec((1, HIDDEN), lambda j, b: (0, 0)),
            pl.BlockSpec((1, HIDDEN), lambda j, b: (0, 0)),
        ],
        out_specs=pl.BlockSpec((TBLK, HIDDEN), lambda j, b: (b * wpb + j, 0)),
        out_shape=jax.ShapeDtypeStruct((N, HIDDEN), jnp.float32),
    )(gathered, segf, pos, tte, gam2d, bet2d)


def kernel(token, segment, word_embeddings, token_type_embeddings,
           position_embeddings, ln_gamma, ln_beta):
    tok = token.reshape(N).astype(jnp.int32)
    segf = segment.reshape(N, 1).astype(jnp.float32)
    gathered = _gather_sc(tok, word_embeddings)
    out = _ln_tc(gathered, segf, position_embeddings,
                 token_type_embeddings, ln_gamma.reshape(1, HIDDEN),
                 ln_beta.reshape(1, HIDDEN))
    return out.reshape(B, S, HIDDEN)
